# baseline (speedup 1.0000x reference)
import functools

import jax
import jax.numpy as jnp
from jax import lax
from jax.experimental import pallas as pl
from jax.experimental.pallas import tpu as pltpu
from jax.experimental.pallas import tpu_sc as plsc

_B = 8
_N = 5000
_NP = 5120
_G = 64
_NCORE = 2
_NSUB = 16
_APS = _NP // _NSUB
_NV = _APS // 16
_BSC = 4
_BPC = _BSC // _NCORE
_CPC = _BPC * _G
_THR = 0.5


def _tc_match_body(a_ref, b_ref, c_ref, gt_ref, valid_ref, cls_out, crd_out):
    ax1 = a_ref[0:1, :]
    ay1 = a_ref[1:2, :]
    ax2 = a_ref[2:3, :]
    ay2 = a_ref[3:4, :]
    area_a = (ax2 - ax1) * (ay2 - ay1)
    idx = lax.broadcasted_iota(jnp.int32, (_G, _NP), 1)

    class_acc = jnp.float32(0.0)
    coord_acc = jnp.float32(0.0)
    for i in range(_B - _BSC):
        g = gt_ref[i]
        cx = g[:, 0:1]
        cy = g[:, 1:2]
        hw = g[:, 2:3] * 0.5
        hh = g[:, 3:4] * 0.5
        gx1 = cx - hw
        gy1 = cy - hh
        gx2 = cx + hw
        gy2 = cy + hh
        area_b = (gx2 - gx1) * (gy2 - gy1)

        iw = jnp.maximum(jnp.minimum(ax2, gx2) - jnp.maximum(ax1, gx1), 0.0)
        ih = jnp.maximum(jnp.minimum(ay2, gy2) - jnp.maximum(ay1, gy1), 0.0)
        inter = iw * ih
        iou = inter / (area_a + area_b - inter)

        colmax = jnp.max(iou, axis=1, keepdims=True)
        midx = jnp.where(iou == colmax, idx, _NP)
        minidx = jnp.min(midx, axis=1, keepdims=True)
        forced = idx == minidx

        vb = valid_ref[i] > 0.5
        mask = ((iou > _THR) | forced) & vb
        maskf = mask.astype(jnp.float32)

        bx = b_ref[i]
        d = (jnp.abs(bx[0:1, :] - gx1) + jnp.abs(bx[1:2, :] - gy1)
             + jnp.abs(bx[2:3, :] - gx2) + jnp.abs(bx[3:4, :] - gy2))
        ctot = jnp.sum(maskf * d)
        cnt = jnp.sum(maskf) * 4.0
        coord_acc = coord_acc + ctot / cnt

        pos = jnp.any(mask, axis=0, keepdims=True)
        p = jnp.where(pos, c_ref[i, 1:2, :], c_ref[i, 0:1, :])
        omp = 1.0 - p
        class_acc = class_acc + jnp.sum(-(omp * omp) * jnp.log(p))

    cls_out[0, 0] = class_acc
    crd_out[0, 0] = coord_acc


def _combine_body(sums_ref, pos_ref, c_ref, cls47_ref, crd47_ref,
                  tot_out, cls_out, crd_out):
    class_acc = jnp.float32(0.0)
    for b in range(_BSC):
        pos = pos_ref[b:b + 1, :] > 0.5
        p = jnp.where(pos, c_ref[b, 1:2, :], c_ref[b, 0:1, :])
        omp = 1.0 - p
        class_acc = class_acc + jnp.sum(-(omp * omp) * jnp.log(p))

    s0 = sums_ref[0]
    s1 = sums_ref[1]
    r0 = jnp.sum(s0, axis=0, keepdims=True)
    r1 = jnp.sum(s1, axis=0, keepdims=True)
    tots = jnp.concatenate([r0[:, 0:_BPC], r1[:, 0:_BPC]], axis=0)
    cnts = jnp.concatenate([r0[:, _BPC:2 * _BPC],
                            r1[:, _BPC:2 * _BPC]], axis=0) * 4.0
    coord03 = jnp.sum(tots / cnts)

    cls = (class_acc + cls47_ref[0, 0]) * (0.01 / _B)
    crd = (coord03 + crd47_ref[0, 0]) * (1.0 / _B)
    cls_out[0, 0] = cls
    crd_out[0, 0] = crd
    tot_out[0, 0] = cls + crd


def _sc_body(anc_hbm, box_hbm, gt_hbm, valid_hbm, out_hbm, pos_hbm,
             anc_v, area_v, box_v, gt_v, valid_v, pos_v,
             part_v, all_v, red_v, sums_v, corrv_v,
             sh_parts, sh_red):
    ci = lax.axis_index("c")
    s = lax.axis_index("s")

    pltpu.sync_copy(anc_hbm.at[s], anc_v)
    pltpu.sync_copy(box_hbm.at[ci, s], box_v)
    pltpu.sync_copy(gt_hbm.at[ci], gt_v)
    pltpu.sync_copy(valid_hbm.at[ci], valid_v)

    zeros16 = jnp.zeros((16,), jnp.float32)
    zeros16i = jnp.zeros((16,), jnp.int32)
    lane = lax.iota(jnp.int32, 16)
    for v in range(_NV):
        sl = pl.ds(v * 16, 16)
        ax1 = anc_v[0, sl]
        ay1 = anc_v[1, sl]
        ax2 = anc_v[2, sl]
        ay2 = anc_v[3, sl]
        area_v[sl] = (ax2 - ax1) * (ay2 - ay1)
        for bl in range(_BPC):
            pos_v[pl.ds(bl * _APS + v * 16, 16)] = zeros16

    corrv_v[...] = zeros16

    base_idx = s * _APS
    sums_vec = zeros16

    for bl in range(_BPC):
        def jbody(j, carry, bl=bl):
            ctot, cnt = carry
            c = bl * _G + j
            cidx = jnp.full((16,), c, jnp.int32)
            cx = plsc.load_gather(gt_v, [cidx])
            cy = plsc.load_gather(gt_v, [cidx + _CPC])
            w = plsc.load_gather(gt_v, [cidx + 2 * _CPC])
            h = plsc.load_gather(gt_v, [cidx + 3 * _CPC])
            vf = plsc.load_gather(valid_v, [cidx])
            gx1 = cx - w * 0.5
            gy1 = cy - h * 0.5
            gx2 = cx + w * 0.5
            gy2 = cy + h * 0.5
            ab = w * h
            m_vec = jnp.full((16,), -1.0, jnp.float32)
            i_vec = zeros16i
            d_vec = zeros16
            for v in range(_NV):
                sl = pl.ds(v * 16, 16)
                ax1 = anc_v[0, sl]
                ay1 = anc_v[1, sl]
                ax2 = anc_v[2, sl]
                ay2 = anc_v[3, sl]
                aa = area_v[sl]
                iw = jnp.maximum(jnp.minimum(ax2, gx2) - jnp.maximum(ax1, gx1), 0.0)
                ih = jnp.maximum(jnp.minimum(ay2, gy2) - jnp.maximum(ay1, gy1), 0.0)
                inter = iw * ih
                iou = inter / ((aa + ab) - inter)
                bx1 = box_v[bl, 0, sl]
                by1 = box_v[bl, 1, sl]
                bx2 = box_v[bl, 2, sl]
                by2 = box_v[bl, 3, sl]
                d = (jnp.abs(bx1 - gx1) + jnp.abs(by1 - gy1)
                     + jnp.abs(bx2 - gx2) + jnp.abs(by2 - gy2))
                mf = jnp.where(iou > _THR, vf, 0.0)
                ctot = ctot + mf * d
                cnt = cnt + mf
                slp = pl.ds(bl * _APS + v * 16, 16)
                pos_v[slp] = jnp.maximum(pos_v[slp], mf)
                better = iou > m_vec
                m_vec = jnp.where(better, iou, m_vec)
                i_vec = jnp.where(better, lane + (base_idx + v * 16), i_vec)
                d_vec = jnp.where(better, d, d_vec)
            m = jnp.max(m_vec)
            cand = jnp.where(m_vec == m, i_vec, _NP + 16)
            bi = jnp.min(cand)
            dw = jnp.max(jnp.where(i_vec == bi, d_vec, 0.0))
            vals = jnp.where(lane == 0, m,
                             jnp.where(lane == 1, bi.astype(jnp.float32), dw))
            plsc.store_scatter(part_v, [cidx + jnp.minimum(lane, 2) * _CPC],
                               vals, mask=lane < 3)
            return ctot, cnt
        ctot, cnt = lax.fori_loop(0, _G, jbody, (zeros16, zeros16))
        sums_vec = sums_vec + jnp.where(lane == bl, jnp.sum(ctot), 0.0)
        sums_vec = sums_vec + jnp.where(lane == _BPC + bl, jnp.sum(cnt), 0.0)

    pltpu.sync_copy(part_v, sh_parts.at[s])
    plsc.subcore_barrier()

    @pl.when(s == 0)
    def _():
        pltpu.sync_copy(sh_parts, all_v)
        corr_vec = zeros16
        for k in range(_CPC // 16):
            sl = pl.ds(k * 16, 16)
            slm = pl.ds(k * 16, 16)
            sli = pl.ds(_CPC + k * 16, 16)
            sld = pl.ds(2 * _CPC + k * 16, 16)
            m = all_v[0, slm]
            i = all_v[0, sli]
            d = all_v[0, sld]
            for wkr in range(1, _NSUB):
                mw = all_v[wkr, slm]
                iw_ = all_v[wkr, sli]
                dw_ = all_v[wkr, sld]
                better = (mw > m) | ((mw == m) & (iw_ < i))
                m = jnp.where(better, mw, m)
                i = jnp.where(better, iw_, i)
                d = jnp.where(better, dw_, d)
            red_v[sl] = i
            vv = valid_v[sl]
            newmask = (m <= _THR) & (vv > 0.5)
            bl = k >> 2
            tcor = jnp.sum(jnp.where(newmask, d, 0.0))
            ccor = jnp.sum(jnp.where(newmask, 1.0, 0.0))
            corr_vec = corr_vec + jnp.where(lane == bl, tcor, 0.0)
            corr_vec = corr_vec + jnp.where(lane == _BPC + bl, ccor, 0.0)
        corrv_v[...] = corr_vec
        pltpu.sync_copy(red_v, sh_red)
    plsc.subcore_barrier()

    pltpu.sync_copy(sh_red, red_v)
    ones16 = jnp.ones((16,), jnp.float32)
    for k in range(_CPC // 16):
        sl = pl.ds(k * 16, 16)
        iv = red_v[sl].astype(jnp.int32)
        vv = valid_v[sl]
        inr = (iv >= base_idx) & (iv < base_idx + _APS) & (vv > 0.5)
        local = jnp.clip(iv - base_idx, 0, _APS - 1)
        plsc.store_scatter(pos_v, [local + (k >> 2) * _APS], ones16,
                           mask=inr)

    sums_vec = sums_vec + corrv_v[...]
    sums_v[...] = sums_vec
    pltpu.sync_copy(sums_v, out_hbm.at[ci, s])
    pltpu.sync_copy(pos_v, pos_hbm.at[ci, s])


_sc_match = pl.kernel(
    _sc_body,
    out_type=[jax.ShapeDtypeStruct((_NCORE, _NSUB, 16), jnp.float32),
              jax.ShapeDtypeStruct((_NCORE, _NSUB, _BPC * _APS), jnp.float32)],
    mesh=plsc.VectorSubcoreMesh(core_axis_name="c", subcore_axis_name="s",
                                num_cores=_NCORE, num_subcores=_NSUB),
    compiler_params=pltpu.CompilerParams(needs_layout_passes=False),
    scratch_types=[
        pltpu.VMEM((4, _APS), jnp.float32),
        pltpu.VMEM((_APS,), jnp.float32),
        pltpu.VMEM((_BPC, 4, _APS), jnp.float32),
        pltpu.VMEM((4 * _CPC,), jnp.float32),
        pltpu.VMEM((_CPC,), jnp.float32),
        pltpu.VMEM((_BPC * _APS,), jnp.float32),
        pltpu.VMEM((3 * _CPC,), jnp.float32),
        pltpu.VMEM((_NSUB, 3 * _CPC), jnp.float32),
        pltpu.VMEM((_CPC,), jnp.float32),
        pltpu.VMEM((16,), jnp.float32),
        pltpu.VMEM((16,), jnp.float32),
        pltpu.VMEM_SHARED((_NSUB, 3 * _CPC), jnp.float32),
        pltpu.VMEM_SHARED((_CPC,), jnp.float32),
    ],
)


@jax.jit
def kernel(batch_boxes, batch_classes, anchors, batch_gt, batch_num_objects):
    pad = _NP - _N
    pad_anchor = jnp.tile(
        jnp.array([[-3.0, -3.0, -2.0, -2.0]], dtype=jnp.float32), (pad, 1))
    anchors_p = jnp.concatenate([anchors, pad_anchor], axis=0)
    anchors_t = anchors_p.T
    anc_sc = anchors_t.reshape(4, _NSUB, _APS).transpose(1, 0, 2)

    boxes_p = jnp.concatenate(
        [batch_boxes, jnp.zeros((_B, pad, 4), jnp.float32)], axis=1)
    box_t = boxes_p.transpose(0, 2, 1)
    box_sc = (box_t[:_BSC].reshape(_NCORE, _BPC, 4, _NSUB, _APS)
              .transpose(0, 3, 1, 2, 4))

    gt_t = batch_gt.transpose(0, 2, 1)
    gt_sc = (gt_t[:_BSC].reshape(_NCORE, _BPC, 4, _G)
             .transpose(0, 2, 1, 3).reshape(_NCORE, 4 * _CPC))

    valid = (jnp.arange(_G, dtype=jnp.int32)[None, :]
             < batch_num_objects.astype(jnp.int32)[:, None])
    validf = valid.astype(jnp.float32)
    valid_sc = validf[:_BSC].reshape(_NCORE, _CPC)

    classes_p = jnp.concatenate(
        [batch_classes, jnp.ones((_B, pad, 2), jnp.float32)], axis=1)
    classes_t = classes_p.transpose(0, 2, 1)

    sums, pos = _sc_match(anc_sc, box_sc, gt_sc, valid_sc)

    cls47, crd47 = pl.pallas_call(
        _tc_match_body,
        out_shape=[jax.ShapeDtypeStruct((1, 1), jnp.float32)] * 2,
        out_specs=[pl.BlockSpec(memory_space=pltpu.SMEM)] * 2,
    )(anchors_t, box_t[_BSC:], classes_t[_BSC:], batch_gt[_BSC:],
      validf[_BSC:, :, None])

    pos_b = (pos.reshape(_NCORE, _NSUB, _BPC, _APS)
             .transpose(0, 2, 1, 3).reshape(_BSC, _NP))

    tot, cls, crd = pl.pallas_call(
        _combine_body,
        out_shape=[jax.ShapeDtypeStruct((1, 1), jnp.float32)] * 3,
        in_specs=[pl.BlockSpec(memory_space=pltpu.VMEM)] * 3
        + [pl.BlockSpec(memory_space=pltpu.SMEM)] * 2,
        out_specs=[pl.BlockSpec(memory_space=pltpu.SMEM)] * 3,
    )(sums, pos_b, classes_t[:_BSC], cls47, crd47)
    return (tot.reshape(1), cls.reshape(1), crd.reshape(1))
